# baseline (speedup 1.0000x reference)
import functools

import numpy as np
import jax
import jax.numpy as jnp
from jax import lax
from jax.experimental import pallas as pl
from jax.experimental.pallas import tpu as pltpu
from jax.experimental.pallas import tpu_sc as plsc

_B = 16384
_N_NUM = 13
_N_CAT = 26
_NF = _N_NUM + _N_CAT
_V = 100000
_D = 32
_L = 16

_NC = 2
_NS = 16
_NW = _NC * _NS

_ROWS_W = _B // _NW
_S = 64
_NCHUNK = _ROWS_W // _S
_PAIRS = _S * _N_CAT
_IDXROWS = _PAIRS // _L

_mesh = plsc.VectorSubcoreMesh(core_axis_name="c", subcore_axis_name="s")


@functools.partial(
    pl.kernel,
    mesh=_mesh,
    out_type=jax.ShapeDtypeStruct((_B, _NF, _D), jnp.float32),
    compiler_params=pltpu.CompilerParams(use_tc_tiling_on_sc=False),
    scratch_types=[
        pltpu.VMEM((_PAIRS,), jnp.int32),
        pltpu.VMEM((_PAIRS, _D), jnp.float32),
        pltpu.VMEM((_S * _N_NUM,), jnp.float32),
        pltpu.VMEM((_S, _N_NUM, _D), jnp.float32),
        pltpu.VMEM((_N_NUM, _D), jnp.float32),
        pltpu.VMEM((_N_NUM, _D), jnp.float32),
        pltpu.VMEM((_N_CAT, _D), jnp.float32),
        pltpu.SemaphoreType.DMA,
    ],
)
def _sc_embed(tables_hbm, idx_hbm, num_hbm, w_hbm, bn_hbm, bc_hbm, out_hbm,
              idx_v, rows_v, num_v, numout_v, w_v, bn_v, bc_v, sem):
    wid = lax.axis_index("s") * _NC + lax.axis_index("c")
    pltpu.sync_copy(w_hbm, w_v)
    pltpu.sync_copy(bn_hbm, bn_v)
    pltpu.sync_copy(bc_hbm, bc_v)

    def chunk_body(c, carry):
        b0 = wid * _ROWS_W + c * _S
        r0 = b0 * _N_CAT
        pltpu.sync_copy(idx_hbm.at[pl.ds(r0, _PAIRS)], idx_v)
        lanes = lax.iota(jnp.int32, _L)
        for j in range(_IDXROWS):
            foff = ((lanes + (j * _L)) % _N_CAT) * _V
            sl = pl.ds(j * _L, _L)
            idx_v[sl] = idx_v[sl] + foff
        pltpu.async_copy(tables_hbm.at[idx_v], rows_v, sem).wait()
        pltpu.sync_copy(num_hbm.at[pl.ds(b0 * _N_NUM, _S * _N_NUM)], num_v)

        def row_body(b, inner):
            for i in range(_N_NUM):
                p = b * _N_NUM + i
                base = (p // _L) * _L
                lane = p - base
                vals = num_v[pl.ds(base, _L)]
                val = lax.gather(
                    vals,
                    jnp.full((_L, 1), lane, jnp.int32),
                    lax.GatherDimensionNumbers(
                        offset_dims=(), collapsed_slice_dims=(0,),
                        start_index_map=(0,)),
                    slice_sizes=(1,),
                    mode=lax.GatherScatterMode.PROMISE_IN_BOUNDS)
                for h in range(2):
                    sl = pl.ds(h * _L, _L)
                    numout_v[b, i, sl] = val * w_v[i, sl] + bn_v[i, sl]
            for f in range(_N_CAT):
                r = b * _N_CAT + f
                for h in range(2):
                    sl = pl.ds(h * _L, _L)
                    rows_v[r, sl] = rows_v[r, sl] + bc_v[f, sl]
            return inner

        lax.fori_loop(0, _S, row_body, 0)
        pltpu.sync_copy(numout_v, out_hbm.at[pl.ds(b0, _S), pl.ds(0, _N_NUM)])
        copies = [
            pltpu.async_copy(
                rows_v.at[pl.ds(b * _N_CAT, _N_CAT)],
                out_hbm.at[b0 + b, pl.ds(_N_NUM, _N_CAT)],
                sem,
            )
            for b in range(_S)
        ]
        for cp in copies:
            cp.wait()
        return carry

    lax.fori_loop(0, _NCHUNK, chunk_body, 0)


def kernel(numerical, cat_idx, w_num, b_num, tables, num_token, cat_token,
           pos_enc):
    biasn = b_num + num_token + pos_enc[:_N_NUM]
    biasc = cat_token + pos_enc[_N_NUM:]
    tables_flat = tables.reshape(_N_CAT * _V, _D)
    idx2 = cat_idx.reshape(_B * _N_CAT)
    num_flat = numerical.reshape(_B * _N_NUM)
    return _sc_embed(tables_flat, idx2, num_flat, w_num, biasn, biasc)
